# Initial kernel scaffold; baseline (speedup 1.0000x reference)
import jax
import jax.numpy as jnp
from jax.experimental import pallas as pl


def kernel(xyz, features, params):
    raise NotImplementedError("write your pallas kernel here")


# speedup vs baseline: 7.8990x; 7.8990x over previous
import functools
import jax
import jax.numpy as jnp
from jax.experimental import pallas as pl

_B, _N, _C_IN = 16, 2048, 3
_NPOINTS = (512, 128)
_RADII = (0.2, 0.4)
_NSAMPLES = (32, 64)


def _fps_kernel(x_ref, y_ref, z_ref, ox_ref, oy_ref, oz_ref, *, npoint, n):
    x = x_ref[...]
    y = y_ref[...]
    z = z_ref[...]
    b = x.shape[0]
    col = jax.lax.broadcasted_iota(jnp.int32, (b, n), 1)
    ocol = jax.lax.broadcasted_iota(jnp.int32, (b, npoint), 1)

    def body(i, state):
        dists, far, ox, oy, oz = state
        sel = col == far
        cx = jnp.sum(jnp.where(sel, x, 0.0), axis=1, keepdims=True)
        cy = jnp.sum(jnp.where(sel, y, 0.0), axis=1, keepdims=True)
        cz = jnp.sum(jnp.where(sel, z, 0.0), axis=1, keepdims=True)
        osel = ocol == i
        ox = jnp.where(osel, cx, ox)
        oy = jnp.where(osel, cy, oy)
        oz = jnp.where(osel, cz, oz)
        d = (x - cx) ** 2 + (y - cy) ** 2 + (z - cz) ** 2
        dists = jnp.minimum(dists, d)
        m = jnp.max(dists, axis=1, keepdims=True)
        far = jnp.min(jnp.where(dists == m, col, n), axis=1, keepdims=True)
        return dists, far, ox, oy, oz

    dists0 = jnp.full((b, n), 1e10, jnp.float32)
    far0 = jnp.zeros((b, 1), jnp.int32)
    z0 = jnp.zeros((b, npoint), jnp.float32)
    _, _, ox, oy, oz = jax.lax.fori_loop(0, npoint, body,
                                         (dists0, far0, z0, z0, z0))
    ox_ref[...] = ox
    oy_ref[...] = oy
    oz_ref[...] = oz


def _fps_call(xyz, npoint):
    b, n, _ = xyz.shape
    x = xyz[:, :, 0]
    y = xyz[:, :, 1]
    z = xyz[:, :, 2]
    out = jax.ShapeDtypeStruct((b, npoint), jnp.float32)
    ox, oy, oz = pl.pallas_call(
        functools.partial(_fps_kernel, npoint=npoint, n=n),
        out_shape=(out, out, out),
    )(x, y, z)
    return jnp.stack([ox, oy, oz], axis=-1)


def _lane_cumsum(mask_f, chunk=128):
    rows, n = mask_f.shape
    nchunk = n // chunk
    tri = jnp.triu(jnp.ones((chunk, chunk), jnp.float32))
    local = jnp.dot(mask_f.reshape(rows * nchunk, chunk), tri,
                    preferred_element_type=jnp.float32)
    local = local.reshape(rows, nchunk, chunk)
    totals = local[:, :, chunk - 1]
    tri_ex = jnp.triu(jnp.ones((nchunk, nchunk), jnp.float32), k=1)
    offs = jnp.dot(totals, tri_ex, preferred_element_type=jnp.float32)
    return (local + offs[:, :, None]).reshape(rows, n)


def _group_kernel(c_ref, pt_ref, t_ref, w_ref, y_ref, s_ref, *,
                  r2, nsample, tc, n, c_t, c_out):
    i_t = pl.program_id(1)
    b_i = pl.program_id(0)

    c = c_ref[0]
    pt = pt_ref[0]
    tbl = t_ref[0]
    w = w_ref[...]

    dx = c[:, 0:1] - pt[0:1, :]
    dy = c[:, 1:2] - pt[1:2, :]
    dz = c[:, 2:3] - pt[2:3, :]
    sqd = (dx * dx + dy * dy) + dz * dz
    mask = sqd <= r2
    mask_f = jnp.where(mask, 1.0, 0.0)
    rank = _lane_cumsum(mask_f)
    cnt = rank[:, n - 1:n]
    rankv = jnp.where(mask, rank, 0.0)

    slots = (jax.lax.broadcasted_iota(jnp.int32, (1, nsample, 1), 1)
             .astype(jnp.float32) + 1.0)
    sel = jnp.where(rankv[:, None, :] == slots, 1.0, 0.0)
    g = jnp.dot(sel.reshape(tc * nsample, n), tbl,
                precision=jax.lax.Precision.HIGHEST,
                preferred_element_type=jnp.float32)
    g = g.reshape(tc, nsample, c_t)
    valid = slots <= cnt[:, None, :]
    g = jnp.where(valid, g, g[:, 0:1, :])

    cpad = jnp.concatenate([c, jnp.zeros((tc, c_t - 3), jnp.float32)], axis=1)
    x = (g - cpad[:, None, :]).reshape(tc * nsample, c_t)
    y = jnp.dot(x, w, preferred_element_type=jnp.float32)
    y_ref[0] = y

    p0 = jnp.sum(y, axis=0, keepdims=True)
    p1 = jnp.sum(y * y, axis=0, keepdims=True)
    part = jnp.concatenate([p0, p1, jnp.zeros((6, c_out), jnp.float32)], axis=0)

    @pl.when(jnp.logical_and(b_i == 0, i_t == 0))
    def _():
        s_ref[...] = jnp.zeros_like(s_ref)

    s_ref[...] += part


def _group_call(new_xyz, xyz_t, table, w, radius, nsample, tc):
    b, npoint, _ = new_xyz.shape
    n = xyz_t.shape[2]
    c_t = table.shape[2]
    c_out = w.shape[1]
    grid = (b, npoint // tc)
    y, stats = pl.pallas_call(
        functools.partial(_group_kernel, r2=radius * radius, nsample=nsample,
                          tc=tc, n=n, c_t=c_t, c_out=c_out),
        grid=grid,
        in_specs=[
            pl.BlockSpec((1, tc, 3), lambda bi, ti: (bi, ti, 0)),
            pl.BlockSpec((1, 3, n), lambda bi, ti: (bi, 0, 0)),
            pl.BlockSpec((1, n, c_t), lambda bi, ti: (bi, 0, 0)),
            pl.BlockSpec((c_t, c_out), lambda bi, ti: (0, 0)),
        ],
        out_specs=(
            pl.BlockSpec((1, tc * nsample, c_out), lambda bi, ti: (bi, ti, 0)),
            pl.BlockSpec((8, c_out), lambda bi, ti: (0, 0)),
        ),
        out_shape=(
            jax.ShapeDtypeStruct((b, npoint * nsample, c_out), jnp.float32),
            jax.ShapeDtypeStruct((8, c_out), jnp.float32),
        ),
    )(new_xyz, xyz_t, table, w)
    return y, stats


def _mid_kernel(y_ref, s_ref, w_ref, g_ref, b_ref, o_ref, so_ref, *,
                n_rows, c_out):
    i = pl.program_id(0)
    s = s_ref[...]
    mu = s[0:1, :] / n_rows
    var = s[1:2, :] / n_rows - mu * mu
    h = jnp.maximum(g_ref[...] * (y_ref[...] - mu) / jnp.sqrt(var + 1e-5)
                    + b_ref[...], 0.0)
    o = jnp.dot(h, w_ref[...], preferred_element_type=jnp.float32)
    o_ref[...] = o

    p0 = jnp.sum(o, axis=0, keepdims=True)
    p1 = jnp.sum(o * o, axis=0, keepdims=True)
    part = jnp.concatenate([p0, p1, jnp.zeros((6, c_out), jnp.float32)], axis=0)

    @pl.when(i == 0)
    def _():
        so_ref[...] = jnp.zeros_like(so_ref)

    so_ref[...] += part


def _mid_call(y, stats, w, gam, bet, rows_tile):
    m, c_in = y.shape
    c_out = w.shape[1]
    grid = (m // rows_tile,)
    o, so = pl.pallas_call(
        functools.partial(_mid_kernel, n_rows=float(m), c_out=c_out),
        grid=grid,
        in_specs=[
            pl.BlockSpec((rows_tile, c_in), lambda i: (i, 0)),
            pl.BlockSpec((8, c_in), lambda i: (0, 0)),
            pl.BlockSpec((c_in, c_out), lambda i: (0, 0)),
            pl.BlockSpec((1, c_in), lambda i: (0, 0)),
            pl.BlockSpec((1, c_in), lambda i: (0, 0)),
        ],
        out_specs=(
            pl.BlockSpec((rows_tile, c_out), lambda i: (i, 0)),
            pl.BlockSpec((8, c_out), lambda i: (0, 0)),
        ),
        out_shape=(
            jax.ShapeDtypeStruct((m, c_out), jnp.float32),
            jax.ShapeDtypeStruct((8, c_out), jnp.float32),
        ),
    )(y, stats, w, gam, bet)
    return o, so


def _final_kernel(y_ref, s_ref, g_ref, b_ref, o_ref, *, n_rows, npoint, nsample, c):
    s = s_ref[...]
    mu = s[0:1, :] / n_rows
    var = s[1:2, :] / n_rows - mu * mu
    h = jnp.maximum(g_ref[...] * (y_ref[0] - mu) / jnp.sqrt(var + 1e-5)
                    + b_ref[...], 0.0)
    o_ref[0] = jnp.max(h.reshape(npoint, nsample, c), axis=1)


def _final_call(y, stats, gam, bet, b, npoint, nsample):
    c = y.shape[1]
    m = y.shape[0]
    y3 = y.reshape(b, npoint * nsample, c)
    out = pl.pallas_call(
        functools.partial(_final_kernel, n_rows=float(m), npoint=npoint,
                          nsample=nsample, c=c),
        grid=(b,),
        in_specs=[
            pl.BlockSpec((1, npoint * nsample, c), lambda i: (i, 0, 0)),
            pl.BlockSpec((8, c), lambda i: (0, 0)),
            pl.BlockSpec((1, c), lambda i: (0, 0)),
            pl.BlockSpec((1, c), lambda i: (0, 0)),
        ],
        out_specs=pl.BlockSpec((1, npoint, c), lambda i: (i, 0, 0)),
        out_shape=jax.ShapeDtypeStruct((b, npoint, c), jnp.float32),
    )(y3, stats, gam, bet)
    return out


def _head_kernel(f_ref, gw1_ref, gg1_ref, gb1_ref, gw2_ref, gbias2_ref,
                 gg2_ref, gb2_ref, hw1_ref, hb1_ref, hw2_ref, hb2_ref,
                 o_ref, *, b, npoint):
    f = f_ref[...]
    rows = f.reshape(b * npoint, f.shape[2])
    y = jnp.dot(rows, gw1_ref[...], preferred_element_type=jnp.float32)
    mu = jnp.mean(y, axis=0, keepdims=True)
    var = jnp.mean((y - mu) ** 2, axis=0, keepdims=True)
    h = jnp.maximum(gg1_ref[...] * (y - mu) / jnp.sqrt(var + 1e-5)
                    + gb1_ref[...], 0.0)
    y2 = jnp.dot(h, gw2_ref[...], preferred_element_type=jnp.float32) \
        + gbias2_ref[...]
    mu2 = jnp.mean(y2, axis=0, keepdims=True)
    var2 = jnp.mean((y2 - mu2) ** 2, axis=0, keepdims=True)
    h2 = jnp.maximum(gg2_ref[...] * (y2 - mu2) / jnp.sqrt(var2 + 1e-5)
                     + gb2_ref[...], 0.0)
    gmax = jnp.max(h2.reshape(b, npoint, h2.shape[1]), axis=1)
    h3 = jnp.maximum(jnp.dot(gmax, hw1_ref[...],
                             preferred_element_type=jnp.float32)
                     + hb1_ref[...], 0.0)
    o_ref[...] = jnp.dot(h3, hw2_ref[...],
                         preferred_element_type=jnp.float32) + hb2_ref[...]


def _head_call(f2, params):
    b, npoint, _ = f2.shape
    nclass = params['hw2'].shape[1]
    args = (f2, params['gw1'], params['gg1'].reshape(1, -1),
            params['gb1'].reshape(1, -1), params['gw2'],
            params['gbias2'].reshape(1, -1), params['gg2'].reshape(1, -1),
            params['gb2'].reshape(1, -1), params['hw1'],
            params['hb1'].reshape(1, -1), params['hw2'],
            params['hb2'].reshape(1, -1))
    return pl.pallas_call(
        functools.partial(_head_kernel, b=b, npoint=npoint),
        out_shape=jax.ShapeDtypeStruct((b, nclass), jnp.float32),
    )(*args)


def _sa_level(xyz, feats, layers, npoint, radius, nsample, tc, rows_tile):
    b = xyz.shape[0]
    new_xyz = _fps_call(xyz, npoint)
    xyz_t = jnp.transpose(xyz, (0, 2, 1))
    table = jnp.concatenate([xyz, feats], axis=-1)

    (w1, g1, b1) = layers[0]
    y, st = _group_call(new_xyz, xyz_t, table, w1, radius, nsample, tc)
    m = b * npoint * nsample
    y = y.reshape(m, -1)
    for (w, g, bb) in layers[1:]:
        gam = g1.reshape(1, -1)
        bet = b1.reshape(1, -1)
        y, st_next = _mid_call(y, st, w, gam, bet, rows_tile)
        st = st_next
        g1, b1 = g, bb
    f = _final_call(y, st, g1.reshape(1, -1), b1.reshape(1, -1),
                    b, npoint, nsample)
    return new_xyz, f


def kernel(xyz, features, params):
    x1, f1 = _sa_level(xyz, features, params['sa1'], _NPOINTS[0], _RADII[0],
                       _NSAMPLES[0], tc=64, rows_tile=4096)
    x2, f2 = _sa_level(x1, f1, params['sa2'], _NPOINTS[1], _RADII[1],
                       _NSAMPLES[1], tc=64, rows_tile=4096)
    return _head_call(f2, params)
